# baseline (speedup 1.0000x reference)
import functools

import jax
import jax.numpy as jnp
from jax import lax
from jax.experimental import pallas as pl
from jax.experimental.pallas import tpu as pltpu
from jax.experimental.pallas import tpu_sc as plsc

VOCAB = 1000000
TCB = 512
HALF = TCB * 977
D = 64
NW = 32
BQ = 4096
LQ = 200
BBLK = 128
LCH = 2
CHUNK = BBLK * LCH
NCH = LQ // LCH
LANES = 16


def _tc_pack_kernel(w1_ref, w2_ref, eye_ref, o_ref):
    eye = eye_ref[...]
    dn = (((0,), (0,)), ((), ()))
    o_ref[:, 0:D] = lax.dot_general(
        w1_ref[...] * 8.0, eye, dn, precision=lax.Precision.HIGHEST)
    o_ref[:, D:2 * D] = lax.dot_general(
        w2_ref[...] * 8.0, eye, dn, precision=lax.Precision.HIGHEST)


def _tc_pack(wt):
    grid = HALF // TCB
    eye = jnp.eye(D, dtype=jnp.float32)
    return pl.pallas_call(
        _tc_pack_kernel,
        grid=(grid,),
        in_specs=[
            pl.BlockSpec((D, TCB), lambda c: (0, c)),
            pl.BlockSpec((D, TCB), lambda c: (0, c + HALF // TCB)),
            pl.BlockSpec((D, D), lambda c: (0, 0)),
        ],
        out_specs=pl.BlockSpec((TCB, 2 * D), lambda c: (c, 0)),
        out_shape=jax.ShapeDtypeStruct((HALF, 2 * D), jnp.float32),
    )(wt, wt, eye)


def _sc_gather_build():
    mesh = plsc.VectorSubcoreMesh(core_axis_name="c", subcore_axis_name="s")

    @functools.partial(
        pl.kernel,
        out_type=jax.ShapeDtypeStruct((LQ, D, BQ), jnp.float32),
        mesh=mesh,
        compiler_params=pltpu.CompilerParams(
            use_tc_tiling_on_sc=False, needs_layout_passes=False),
        scratch_types=[
            pltpu.VMEM((CHUNK,), jnp.int32),
            pltpu.VMEM((CHUNK,), jnp.int32),
            pltpu.VMEM((CHUNK,), jnp.int32),
            pltpu.VMEM((CHUNK,), jnp.int32),
            pltpu.VMEM((CHUNK, D), jnp.float32),
            pltpu.VMEM((CHUNK, D), jnp.float32),
            pltpu.VMEM((LCH * D, BBLK), jnp.float32),
            pltpu.VMEM((LCH * D, BBLK), jnp.float32),
            pltpu.SemaphoreType.DMA,
            pltpu.SemaphoreType.DMA,
            pltpu.SemaphoreType.DMA,
            pltpu.SemaphoreType.DMA,
            pltpu.SemaphoreType.DMA,
            pltpu.SemaphoreType.DMA,
        ],
    )
    def emb_kernel(xp_hbm, tab_hbm, out_hbm, idx0, idx1, kx0, kx1, gb0, gb1,
                   st0, st1, si0, si1, sg0, sg1, sw0, sw1):
        wid = lax.axis_index("s") * 2 + lax.axis_index("c")
        idx_v = (idx0, idx1)
        kx = (kx0, kx1)
        gb = (gb0, gb1)
        stag = (st0, st1)
        si = (si0, si1)
        sg = (sg0, sg1)
        sw = (sw0, sw1)

        def start_fetch(g, p):
            pltpu.async_copy(xp_hbm.at[wid, g], idx_v[p], si[p])

        def wait_fetch(g, p):
            pltpu.make_async_copy(xp_hbm.at[wid, g], idx_v[p], si[p]).wait()

        def to_view_rows(p):
            def tbody(q, c):
                sl = pl.ds(q * LANES, LANES)
                iv = idx_v[p][sl]
                kx[p][sl] = iv * 2 - jnp.where(iv >= HALF, 2 * HALF - 1, 0)
                return c

            lax.fori_loop(0, CHUNK // LANES, tbody, 0)

        def start_gather(p):
            pltpu.async_copy(tab_hbm.at[kx[p]], gb[p], sg[p])

        def wait_gather(p):
            pltpu.make_async_copy(tab_hbm.at[kx[p]], gb[p], sg[p]).wait()

        adr = [
            lax.iota(jnp.int32, LANES) + j * LANES
            for j in range(D // LANES)
        ]

        def stage_chunk(p, q):
            buf = gb[p]
            st = stag[q]

            def grp_body(g16, c):
                r0 = g16 * LANES
                l64 = (r0 // BBLK) * D
                rows = [a + l64 for a in adr]
                rb0 = r0 % BBLK
                for k in range(LANES):
                    col = jnp.full((LANES,), rb0 + k, jnp.int32)
                    for j in range(D // LANES):
                        v = buf[r0 + k, pl.ds(j * LANES, LANES)]
                        plsc.store_scatter(st, [rows[j], col], v)
                return c

            lax.fori_loop(0, CHUNK // LANES, grp_body, 0)

        def _wb_pairs(g, q):
            return [
                (stag[q].at[pl.ds(l * D, D)],
                 out_hbm.at[g * LCH + l, :, pl.ds(wid * BBLK, BBLK)])
                for l in range(LCH)
            ]

        def start_wb(g, q):
            for src, dst in _wb_pairs(g, q):
                pltpu.async_copy(src, dst, sw[q])

        def wait_wb(g, q):
            for src, dst in _wb_pairs(g, q):
                pltpu.make_async_copy(src, dst, sw[q]).wait()

        def step(g, p):
            g1 = jnp.minimum(g + 1, NCH - 1)
            g2 = jnp.minimum(g + 2, NCH - 1)
            wait_fetch(g1, 1 - p)
            to_view_rows(1 - p)
            start_gather(1 - p)
            wait_gather(p)

            @pl.when(g >= 2)
            def _():
                wait_wb(g - 2, p)

            stage_chunk(p, p)
            start_fetch(g2, p)
            start_wb(g, p)

        start_fetch(0, 0)
        wait_fetch(0, 0)
        to_view_rows(0)
        start_gather(0)
        start_fetch(1, 1)

        def group_body(t, c):
            step(2 * t, 0)
            step(2 * t + 1, 1)
            return c

        lax.fori_loop(0, NCH // 2, group_body, 0)
        wait_wb(NCH - 2, 0)
        wait_wb(NCH - 1, 1)
        wait_gather(0)
        wait_fetch(NCH - 1, 1)

    return emb_kernel


def kernel(x, embedding_weight):
    tab = _tc_pack(embedding_weight.T)
    tabv = tab.reshape(2 * HALF, D)
    xp = (x.astype(jnp.int32)
          .reshape(NW, BBLK, NCH, LCH)
          .transpose(0, 2, 3, 1)
          .reshape(NW, NCH, CHUNK))
    o = _sc_gather_build()(xp, tabv)
    return jnp.transpose(o, (2, 0, 1))


# speedup vs baseline: 1.0802x; 1.0802x over previous
import functools

import jax
import jax.numpy as jnp
from jax import lax
from jax.experimental import pallas as pl
from jax.experimental.pallas import tpu as pltpu
from jax.experimental.pallas import tpu_sc as plsc

VOCAB = 1000000
TCB = 512
HALF = TCB * 977
D = 64
NW = 32
BQ = 4096
LQ = 200
BBLK = 128
LCH = 2
CHUNK = BBLK * LCH
NCH = LQ // LCH
LANES = 16


def _tc_pack_kernel(w1_ref, w2_ref, eye_ref, o_ref):
    eye = eye_ref[...]
    dn = (((0,), (0,)), ((), ()))
    o_ref[:, 0:D] = lax.dot_general(w1_ref[...] * 8.0, eye, dn)
    o_ref[:, D:2 * D] = lax.dot_general(w2_ref[...] * 8.0, eye, dn)


def _tc_pack(wt):
    grid = HALF // TCB
    eye = jnp.eye(D, dtype=jnp.float32)
    return pl.pallas_call(
        _tc_pack_kernel,
        grid=(grid,),
        in_specs=[
            pl.BlockSpec((D, TCB), lambda c: (0, c)),
            pl.BlockSpec((D, TCB), lambda c: (0, c + HALF // TCB)),
            pl.BlockSpec((D, D), lambda c: (0, 0)),
        ],
        out_specs=pl.BlockSpec((TCB, 2 * D), lambda c: (c, 0)),
        out_shape=jax.ShapeDtypeStruct((HALF, 2 * D), jnp.float32),
    )(wt, wt, eye)


def _sc_gather_build():
    mesh = plsc.VectorSubcoreMesh(core_axis_name="c", subcore_axis_name="s")

    @functools.partial(
        pl.kernel,
        out_type=jax.ShapeDtypeStruct((LQ, D, BQ), jnp.float32),
        mesh=mesh,
        compiler_params=pltpu.CompilerParams(
            use_tc_tiling_on_sc=False, needs_layout_passes=False),
        scratch_types=[
            pltpu.VMEM((CHUNK,), jnp.int32),
            pltpu.VMEM((CHUNK,), jnp.int32),
            pltpu.VMEM((CHUNK,), jnp.int32),
            pltpu.VMEM((CHUNK,), jnp.int32),
            pltpu.VMEM((CHUNK, D), jnp.float32),
            pltpu.VMEM((CHUNK, D), jnp.float32),
            pltpu.VMEM((LCH * D, BBLK), jnp.float32),
            pltpu.VMEM((LCH * D, BBLK), jnp.float32),
            pltpu.SemaphoreType.DMA,
            pltpu.SemaphoreType.DMA,
            pltpu.SemaphoreType.DMA,
            pltpu.SemaphoreType.DMA,
            pltpu.SemaphoreType.DMA,
            pltpu.SemaphoreType.DMA,
        ],
    )
    def emb_kernel(xp_hbm, tab_hbm, out_hbm, idx0, idx1, kx0, kx1, gb0, gb1,
                   st0, st1, si0, si1, sg0, sg1, sw0, sw1):
        wid = lax.axis_index("s") * 2 + lax.axis_index("c")
        idx_v = (idx0, idx1)
        kx = (kx0, kx1)
        gb = (gb0, gb1)
        stag = (st0, st1)
        si = (si0, si1)
        sg = (sg0, sg1)
        sw = (sw0, sw1)

        def start_fetch(g, p):
            pltpu.async_copy(xp_hbm.at[wid, g], idx_v[p], si[p])

        def wait_fetch(g, p):
            pltpu.make_async_copy(xp_hbm.at[wid, g], idx_v[p], si[p]).wait()

        def to_view_rows(p):
            def tbody(q, c):
                sl = pl.ds(q * LANES, LANES)
                iv = idx_v[p][sl]
                kx[p][sl] = iv * 2 - jnp.where(iv >= HALF, 2 * HALF - 1, 0)
                return c

            lax.fori_loop(0, CHUNK // LANES, tbody, 0)

        def start_gather(p):
            pltpu.async_copy(tab_hbm.at[kx[p]], gb[p], sg[p])

        def wait_gather(p):
            pltpu.make_async_copy(tab_hbm.at[kx[p]], gb[p], sg[p]).wait()

        adr = [
            lax.iota(jnp.int32, LANES) + j * LANES
            for j in range(D // LANES)
        ]

        def stage_chunk(p, q):
            buf = gb[p]
            st = stag[q]

            def grp_body(g16, c):
                r0 = g16 * LANES
                l64 = (r0 // BBLK) * D
                rows = [a + l64 for a in adr]
                rb0 = r0 % BBLK
                for k in range(LANES):
                    col = jnp.full((LANES,), rb0 + k, jnp.int32)
                    for j in range(D // LANES):
                        v = buf[r0 + k, pl.ds(j * LANES, LANES)]
                        plsc.store_scatter(st, [rows[j], col], v)
                return c

            lax.fori_loop(0, CHUNK // LANES, grp_body, 0, unroll=2)

        def _wb_pairs(g, q):
            return [
                (stag[q].at[pl.ds(l * D, D)],
                 out_hbm.at[g * LCH + l, :, pl.ds(wid * BBLK, BBLK)])
                for l in range(LCH)
            ]

        def start_wb(g, q):
            for src, dst in _wb_pairs(g, q):
                pltpu.async_copy(src, dst, sw[q])

        def wait_wb(g, q):
            for src, dst in _wb_pairs(g, q):
                pltpu.make_async_copy(src, dst, sw[q]).wait()

        def step(g, p):
            g1 = jnp.minimum(g + 1, NCH - 1)
            g2 = jnp.minimum(g + 2, NCH - 1)
            wait_fetch(g1, 1 - p)
            to_view_rows(1 - p)
            start_gather(1 - p)
            wait_gather(p)

            @pl.when(g >= 2)
            def _():
                wait_wb(g - 2, p)

            stage_chunk(p, p)
            start_fetch(g2, p)
            start_wb(g, p)

        start_fetch(0, 0)
        wait_fetch(0, 0)
        to_view_rows(0)
        start_gather(0)
        start_fetch(1, 1)

        def group_body(t, c):
            step(2 * t, 0)
            step(2 * t + 1, 1)
            return c

        lax.fori_loop(0, NCH // 2, group_body, 0)
        wait_wb(NCH - 2, 0)
        wait_wb(NCH - 1, 1)
        wait_gather(0)
        wait_fetch(NCH - 1, 1)

    return emb_kernel


def kernel(x, embedding_weight):
    tab = _tc_pack(embedding_weight.T)
    tabv = tab.reshape(2 * HALF, D)
    xp = (x.astype(jnp.int32)
          .reshape(NW, BBLK, NCH, LCH)
          .transpose(0, 2, 3, 1)
          .reshape(NW, NCH, CHUNK))
    o = _sc_gather_build()(xp, tabv)
    return jnp.transpose(o, (2, 0, 1))


# speedup vs baseline: 1.5754x; 1.4585x over previous
import functools

import jax
import jax.numpy as jnp
from jax import lax
from jax.experimental import pallas as pl
from jax.experimental.pallas import tpu as pltpu
from jax.experimental.pallas import tpu_sc as plsc

VOCAB = 1000000
TCB = 512
HALF = TCB * 977
D = 64
NW = 32
BQ = 4096
LQ = 200
NCH = BQ // NW
LANES = 16
IPAD = 208


def _tc_pack_kernel(w1_ref, w2_ref, eye_ref, o_ref):
    eye = eye_ref[...]
    dn = (((0,), (0,)), ((), ()))
    o_ref[:, 0:D] = lax.dot_general(w1_ref[...] * 8.0, eye, dn)
    o_ref[:, D:2 * D] = lax.dot_general(w2_ref[...] * 8.0, eye, dn)


def _tc_pack(wt):
    grid = HALF // TCB
    eye = jnp.eye(D, dtype=jnp.float32)
    return pl.pallas_call(
        _tc_pack_kernel,
        grid=(grid,),
        in_specs=[
            pl.BlockSpec((D, TCB), lambda c: (0, c)),
            pl.BlockSpec((D, TCB), lambda c: (0, c + HALF // TCB)),
            pl.BlockSpec((D, D), lambda c: (0, 0)),
        ],
        out_specs=pl.BlockSpec((TCB, 2 * D), lambda c: (c, 0)),
        out_shape=jax.ShapeDtypeStruct((HALF, 2 * D), jnp.float32),
    )(wt, wt, eye)


def _sc_gather_build():
    mesh = plsc.VectorSubcoreMesh(core_axis_name="c", subcore_axis_name="s")

    @functools.partial(
        pl.kernel,
        out_type=jax.ShapeDtypeStruct((BQ, LQ, D), jnp.float32),
        mesh=mesh,
        compiler_params=pltpu.CompilerParams(use_tc_tiling_on_sc=False),
        scratch_types=[
            pltpu.VMEM((IPAD,), jnp.int32),
            pltpu.VMEM((IPAD,), jnp.int32),
            pltpu.VMEM((IPAD,), jnp.int32),
            pltpu.VMEM((IPAD,), jnp.int32),
            pltpu.VMEM((LQ, D), jnp.float32),
            pltpu.VMEM((LQ, D), jnp.float32),
            pltpu.SemaphoreType.DMA,
            pltpu.SemaphoreType.DMA,
            pltpu.SemaphoreType.DMA,
            pltpu.SemaphoreType.DMA,
            pltpu.SemaphoreType.DMA,
            pltpu.SemaphoreType.DMA,
        ],
    )
    def emb_kernel(xp_hbm, tab_hbm, out_hbm, idx0, idx1, kx0, kx1, gb0, gb1,
                   si0, si1, sg0, sg1, sw0, sw1):
        wid = lax.axis_index("s") * 2 + lax.axis_index("c")
        b0 = wid * NCH
        idx_v = (idx0, idx1)
        kx = (kx0, kx1)
        gb = (gb0, gb1)
        si = (si0, si1)
        sg = (sg0, sg1)
        sw = (sw0, sw1)

        def start_fetch(g, p):
            pltpu.async_copy(
                xp_hbm.at[wid, g], idx_v[p].at[pl.ds(0, LQ)], si[p])

        def wait_fetch(g, p):
            pltpu.make_async_copy(
                xp_hbm.at[wid, g], idx_v[p].at[pl.ds(0, LQ)], si[p]).wait()

        def to_view_rows(p):
            def tbody(q, c):
                sl = pl.ds(q * LANES, LANES)
                iv = idx_v[p][sl]
                kx[p][sl] = iv * 2 - jnp.where(iv >= HALF, 2 * HALF - 1, 0)
                return c

            lax.fori_loop(0, IPAD // LANES, tbody, 0)

        def start_gather(p):
            pltpu.async_copy(
                tab_hbm.at[kx[p].at[pl.ds(0, LQ)]], gb[p], sg[p])

        def wait_gather(p):
            pltpu.make_async_copy(
                tab_hbm.at[kx[p].at[pl.ds(0, LQ)]], gb[p], sg[p]).wait()

        def start_wb(g, p):
            pltpu.async_copy(gb[p], out_hbm.at[b0 + g], sw[p])

        def wait_wb(g, p):
            pltpu.make_async_copy(gb[p], out_hbm.at[b0 + g], sw[p]).wait()

        def step(g, p):
            g1 = jnp.minimum(g + 1, NCH - 1)
            g2 = jnp.minimum(g + 2, NCH - 1)
            wait_fetch(g1, 1 - p)
            to_view_rows(1 - p)

            @pl.when(g >= 1)
            def _():
                wait_wb(g - 1, 1 - p)

            start_gather(1 - p)
            wait_gather(p)
            start_fetch(g2, p)
            start_wb(g, p)

        start_fetch(0, 0)
        wait_fetch(0, 0)
        to_view_rows(0)
        start_gather(0)
        start_fetch(1, 1)

        def group_body(t, c):
            step(2 * t, 0)
            step(2 * t + 1, 1)
            return c

        lax.fori_loop(0, NCH // 2, group_body, 0)
        wait_wb(NCH - 1, 1)
        wait_gather(0)
        wait_fetch(NCH - 1, 1)

    return emb_kernel


def kernel(x, embedding_weight):
    tab = _tc_pack(embedding_weight.T)
    tabv = tab.reshape(2 * HALF, D)
    xp = x.astype(jnp.int32).reshape(NW, NCH, LQ)
    return _sc_gather_build()(xp, tabv)


# speedup vs baseline: 2.0287x; 1.2877x over previous
import functools

import jax
import jax.numpy as jnp
from jax import lax
from jax.experimental import pallas as pl
from jax.experimental.pallas import tpu as pltpu
from jax.experimental.pallas import tpu_sc as plsc

VOCAB = 1000000
TCB = 512
HALF = TCB * 977
D = 64
NW = 32
BQ = 4096
LQ = 200
NCH = BQ // NW
LANES = 16
IPAD = 208


def _tc_pack_kernel(w1_ref, w2_ref, eye_ref, o_ref):
    eye = eye_ref[...]
    dn = (((0,), (0,)), ((), ()))
    o_ref[:, 0:D] = lax.dot_general(w1_ref[...] * 8.0, eye, dn)
    o_ref[:, D:2 * D] = lax.dot_general(w2_ref[...] * 8.0, eye, dn)


def _tc_pack(wt):
    grid = HALF // TCB
    eye = jnp.eye(D, dtype=jnp.float32)
    return pl.pallas_call(
        _tc_pack_kernel,
        grid=(grid,),
        in_specs=[
            pl.BlockSpec((D, TCB), lambda c: (0, c)),
            pl.BlockSpec((D, TCB), lambda c: (0, c + HALF // TCB)),
            pl.BlockSpec((D, D), lambda c: (0, 0)),
        ],
        out_specs=pl.BlockSpec((TCB, 2 * D), lambda c: (c, 0)),
        out_shape=jax.ShapeDtypeStruct((HALF, 2 * D), jnp.float32),
    )(wt, wt, eye)


def _sc_gather_build():
    mesh = plsc.VectorSubcoreMesh(core_axis_name="c", subcore_axis_name="s")

    @functools.partial(
        pl.kernel,
        out_type=jax.ShapeDtypeStruct((BQ, LQ, 2 * D), jnp.float32),
        mesh=mesh,
        compiler_params=pltpu.CompilerParams(use_tc_tiling_on_sc=False),
        scratch_types=[
            pltpu.VMEM((IPAD,), jnp.int32),
            pltpu.VMEM((IPAD,), jnp.int32),
            pltpu.VMEM((IPAD,), jnp.int32),
            pltpu.VMEM((IPAD,), jnp.int32),
            pltpu.VMEM((LQ, D), jnp.float32),
            pltpu.VMEM((LQ, D), jnp.float32),
            pltpu.SemaphoreType.DMA,
            pltpu.SemaphoreType.DMA,
            pltpu.SemaphoreType.DMA,
            pltpu.SemaphoreType.DMA,
            pltpu.SemaphoreType.DMA,
            pltpu.SemaphoreType.DMA,
        ],
    )
    def emb_kernel(xp_hbm, tab_hbm, out_hbm, idx0, idx1, kx0, kx1, gb0, gb1,
                   si0, si1, sg0, sg1, sw0, sw1):
        wid = lax.axis_index("s") * 2 + lax.axis_index("c")
        b0 = wid * NCH
        idx_v = (idx0, idx1)
        kx = (kx0, kx1)
        gb = (gb0, gb1)
        si = (si0, si1)
        sg = (sg0, sg1)
        sw = (sw0, sw1)

        def start_fetch(g, p):
            pltpu.async_copy(
                xp_hbm.at[wid, g], idx_v[p].at[pl.ds(0, LQ)], si[p])

        def wait_fetch(g, p):
            pltpu.make_async_copy(
                xp_hbm.at[wid, g], idx_v[p].at[pl.ds(0, LQ)], si[p]).wait()

        def to_view_rows(p):
            def tbody(q, c):
                sl = pl.ds(q * LANES, LANES)
                iv = idx_v[p][sl]
                kx[p][sl] = iv * 2 - jnp.where(iv >= HALF, 2 * HALF - 1, 0)
                return c

            lax.fori_loop(0, IPAD // LANES, tbody, 0)

        def start_gather(p):
            pltpu.async_copy(
                tab_hbm.at[kx[p].at[pl.ds(0, LQ)]], gb[p], sg[p])

        def wait_gather(p):
            pltpu.make_async_copy(
                tab_hbm.at[kx[p].at[pl.ds(0, LQ)]], gb[p], sg[p]).wait()

        def start_wb(g, p):
            pltpu.async_copy(
                gb[p], out_hbm.at[b0 + g, :, pl.ds(0, D)], sw[p])

        def wait_wb(g, p):
            pltpu.make_async_copy(
                gb[p], out_hbm.at[b0 + g, :, pl.ds(0, D)], sw[p]).wait()

        def step(g, p):
            g1 = jnp.minimum(g + 1, NCH - 1)
            g2 = jnp.minimum(g + 2, NCH - 1)
            wait_fetch(g1, 1 - p)
            to_view_rows(1 - p)

            @pl.when(g >= 1)
            def _():
                wait_wb(g - 1, 1 - p)

            start_gather(1 - p)
            wait_gather(p)
            start_fetch(g2, p)
            start_wb(g, p)

        start_fetch(0, 0)
        wait_fetch(0, 0)
        to_view_rows(0)
        start_gather(0)
        start_fetch(1, 1)

        def group_body(t, c):
            step(2 * t, 0)
            step(2 * t + 1, 1)
            return c

        lax.fori_loop(0, NCH // 2, group_body, 0)
        wait_wb(NCH - 1, 1)
        wait_gather(0)
        wait_fetch(NCH - 1, 1)

    return emb_kernel


def kernel(x, embedding_weight):
    tab = _tc_pack(embedding_weight.T)
    tabv = tab.reshape(2 * HALF, D)
    xp = x.astype(jnp.int32).reshape(NW, NCH, LQ)
    o128 = _sc_gather_build()(xp, tabv)
    return o128[:, :, :D]
